# Initial kernel scaffold; baseline (speedup 1.0000x reference)
import jax
import jax.numpy as jnp
from jax.experimental import pallas as pl


def kernel(feature, edge_index, edge_type, W_in, b_in, W1, b1, W2, b2, Wo, bo):
    raise NotImplementedError("write your pallas kernel here")


# speedup vs baseline: 4.5356x; 4.5356x over previous
import functools

import jax
import jax.numpy as jnp
from jax import lax
from jax.experimental import pallas as pl
from jax.experimental.pallas import tpu as pltpu
from jax.experimental.pallas import tpu_sc as plsc

NC = 2
NS = 16
NW = NC * NS
BATCH = 128
NBUF = 3


def _segment_sum_sc(x2n, src3d, dst3d, acc_rows, nb):
    Dh = x2n.shape[1]
    stripe = acc_rows // NS
    mesh = plsc.VectorSubcoreMesh(core_axis_name="c", subcore_axis_name="s")

    @functools.partial(
        pl.kernel,
        out_type=jax.ShapeDtypeStruct((NC, acc_rows, Dh), jnp.float32),
        mesh=mesh,
        compiler_params=pltpu.CompilerParams(use_tc_tiling_on_sc=False),
        scratch_types=[
            pltpu.VMEM((nb, BATCH), jnp.int32),
            pltpu.VMEM((nb, BATCH), jnp.int32),
            pltpu.VMEM((NBUF, BATCH, Dh), jnp.float32),
            pltpu.VMEM_SHARED((acc_rows, Dh), jnp.float32),
            pltpu.SemaphoreType.DMA,
            pltpu.SemaphoreType.DMA,
            pltpu.SemaphoreType.DMA,
        ],
    )
    def seg_kernel(x_hbm, src_hbm, dst_hbm, zero_hbm, out_hbm,
                   src_v, dst_v, rows_v, acc, sem0, sem1, sem2):
        cid = lax.axis_index("c")
        sid = lax.axis_index("s")
        sems = (sem0, sem1, sem2)

        pltpu.sync_copy(src_hbm.at[cid, sid], src_v)
        pltpu.sync_copy(dst_hbm.at[sid], dst_v)

        pltpu.sync_copy(zero_hbm, acc.at[pl.ds(sid * stripe, stripe)])
        plsc.subcore_barrier()

        for b in range(NBUF):
            pltpu.async_copy(x_hbm.at[src_v.at[b]], rows_v.at[b], sems[b])

        def step(i, _):
            for b in range(NBUF):
                j = i * NBUF + b
                pltpu.make_async_copy(
                    x_hbm.at[src_v.at[j]], rows_v.at[b], sems[b]).wait()
                pltpu.sync_copy(rows_v.at[b], acc.at[dst_v.at[j]], add=True)

                @pl.when(j + NBUF < nb)
                def _():
                    pltpu.async_copy(
                        x_hbm.at[src_v.at[j + NBUF]], rows_v.at[b], sems[b])
            return 0

        lax.fori_loop(0, nb // NBUF, step, 0)
        plsc.subcore_barrier()

        pltpu.sync_copy(acc.at[pl.ds(sid * stripe, stripe)],
                        out_hbm.at[cid, pl.ds(sid * stripe, stripe)])

    zero = jnp.zeros((stripe, Dh), jnp.float32)
    return seg_kernel(x2n, src3d, dst3d, zero)


def _tc_in(feat_ref, w_ref, b_ref, o_ref):
    y = jnp.dot(feat_ref[...], w_ref[...],
                preferred_element_type=jnp.float32) + b_ref[...]
    o_ref[...] = jnp.where(y >= 0, y, 0.01 * y)


def _tc_mid(x_ref, plo_ref, phi_ref, w_ref, b_ref, o_ref):
    agg = jnp.concatenate([plo_ref[0], phi_ref[0]], axis=-1)
    h = x_ref[...] + agg
    o_ref[...] = jnp.dot(h, w_ref[...],
                         preferred_element_type=jnp.float32) + b_ref[...]


def _tc_out(x_ref, plo_ref, phi_ref, w_ref, b_ref, wo_ref, bo_ref, o_ref):
    agg = jnp.concatenate([plo_ref[0], phi_ref[0]], axis=-1)
    h = x_ref[...] + agg
    x2 = jnp.dot(h, w_ref[...],
                 preferred_element_type=jnp.float32) + b_ref[...]
    o_ref[...] = jnp.dot(x2, wo_ref[...],
                         preferred_element_type=jnp.float32) + bo_ref[...]


def kernel(feature, edge_index, edge_type, W_in, b_in, W1, b1, W2, b2, Wo, bo):
    del edge_type
    N, D_in = feature.shape
    D = W_in.shape[1]
    Dh = D // NC
    E = edge_index.shape[1]
    BM = 1000
    grid = (N // BM,)

    nb = -(-E // (NS * BATCH))
    nb = -(-nb // NBUF) * NBUF
    e_pad = NS * BATCH * nb - E
    acc_rows = -(-(N + 1) // (NS * 8)) * (NS * 8)
    pad_dst = N
    src = edge_index[0].astype(jnp.int32)
    dst = edge_index[1].astype(jnp.int32)
    src_p = jnp.concatenate([src, jnp.zeros((e_pad,), jnp.int32)])
    src3d = jnp.stack([2 * src_p, 2 * src_p + 1]).reshape(NC, NS, nb, BATCH)
    dst3d = jnp.concatenate(
        [dst, jnp.full((e_pad,), pad_dst, jnp.int32)]).reshape(NS, nb, BATCH)

    b_in2 = b_in.reshape(1, D)
    b12 = b1.reshape(1, D)
    b22 = b2.reshape(1, D)
    Wo_p = jnp.pad(Wo, ((0, 0), (0, D - Wo.shape[1])))
    bo_p = jnp.pad(bo, (0, D - bo.shape[0])).reshape(1, D)

    row_spec = pl.BlockSpec((BM, D), lambda i: (i, 0))
    plo_spec = pl.BlockSpec((1, BM, Dh), lambda i: (0, i, 0))
    phi_spec = pl.BlockSpec((1, BM, Dh), lambda i: (1, i, 0))
    w_spec = pl.BlockSpec((D, D), lambda i: (0, 0))
    b_spec = pl.BlockSpec((1, D), lambda i: (0, 0))

    x0 = pl.pallas_call(
        _tc_in,
        grid=grid,
        in_specs=[pl.BlockSpec((BM, D_in), lambda i: (i, 0)), w_spec, b_spec],
        out_specs=row_spec,
        out_shape=jax.ShapeDtypeStruct((N, D), jnp.float32),
    )(feature, W_in, b_in2)

    p = _segment_sum_sc(x0.reshape(NC * N, Dh), src3d, dst3d, acc_rows, nb)

    x1 = pl.pallas_call(
        _tc_mid,
        grid=grid,
        in_specs=[row_spec, plo_spec, phi_spec, w_spec, b_spec],
        out_specs=row_spec,
        out_shape=jax.ShapeDtypeStruct((N, D), jnp.float32),
    )(x0, p, p, W1, b12)

    q = _segment_sum_sc(x1.reshape(NC * N, Dh), src3d, dst3d, acc_rows, nb)

    out = pl.pallas_call(
        _tc_out,
        grid=grid,
        in_specs=[row_spec, plo_spec, phi_spec, w_spec, b_spec,
                  w_spec, b_spec],
        out_specs=row_spec,
        out_shape=jax.ShapeDtypeStruct((N, D), jnp.float32),
    )(x1, q, q, W2, b22, Wo_p, bo_p)

    return out[:, :Wo.shape[1]]


# speedup vs baseline: 5.4052x; 1.1917x over previous
import functools

import jax
import jax.numpy as jnp
from jax import lax
from jax.experimental import pallas as pl
from jax.experimental.pallas import tpu as pltpu
from jax.experimental.pallas import tpu_sc as plsc

NC = 2
NS = 16
NW = NC * NS
BATCH = 128
NBUF = 2


def _segment_sum_sc(x2n, src3d, dst3d, acc_rows, nb):
    Dh = x2n.shape[1]
    stripe = acc_rows // NS
    mesh = plsc.VectorSubcoreMesh(core_axis_name="c", subcore_axis_name="s")

    @functools.partial(
        pl.kernel,
        out_type=jax.ShapeDtypeStruct((NC, acc_rows, Dh), jnp.float32),
        mesh=mesh,
        compiler_params=pltpu.CompilerParams(use_tc_tiling_on_sc=False),
        scratch_types=[
            pltpu.VMEM((nb, BATCH), jnp.int32),
            pltpu.VMEM((nb, BATCH), jnp.int32),
            pltpu.VMEM((NBUF, BATCH, Dh), jnp.float32),
            pltpu.VMEM_SHARED((acc_rows, Dh), jnp.float32),
            [pltpu.SemaphoreType.DMA] * NBUF,
        ],
    )
    def seg_kernel(x_hbm, src_hbm, dst_hbm, zero_hbm, out_hbm,
                   src_v, dst_v, rows_v, acc, sem_g):
        cid = lax.axis_index("c")
        sid = lax.axis_index("s")

        pltpu.sync_copy(src_hbm.at[cid, sid], src_v)
        pltpu.sync_copy(dst_hbm.at[sid], dst_v)

        pltpu.sync_copy(zero_hbm, acc.at[pl.ds(sid * stripe, stripe)])
        plsc.subcore_barrier()

        def start_g(j, b):
            pltpu.async_copy(x_hbm.at[src_v.at[j]], rows_v.at[b], sem_g[b])

        def wait_g(j, b):
            pltpu.make_async_copy(
                x_hbm.at[src_v.at[j]], rows_v.at[b], sem_g[b]).wait()

        for b in range(NBUF):
            start_g(b, b)

        def step(i, _):
            for b in range(NBUF):
                j = i * NBUF + b
                wait_g(j, b)
                pltpu.sync_copy(rows_v.at[b], acc.at[dst_v.at[j]], add=True)

                @pl.when(j + NBUF < nb)
                def _():
                    start_g(j + NBUF, b)
            return 0

        lax.fori_loop(0, nb // NBUF, step, 0)
        plsc.subcore_barrier()

        pltpu.sync_copy(acc.at[pl.ds(sid * stripe, stripe)],
                        out_hbm.at[cid, pl.ds(sid * stripe, stripe)])

    zero = jnp.zeros((stripe, Dh), jnp.float32)
    return seg_kernel(x2n, src3d, dst3d, zero)


def _tc_in(feat_ref, w_ref, b_ref, o_ref):
    y = jnp.dot(feat_ref[...], w_ref[...],
                preferred_element_type=jnp.float32) + b_ref[...]
    o_ref[...] = jnp.where(y >= 0, y, 0.01 * y)


def _tc_mid(x_ref, plo_ref, phi_ref, w_ref, b_ref, o_ref):
    agg = jnp.concatenate([plo_ref[0], phi_ref[0]], axis=-1)
    h = x_ref[...] + agg
    o_ref[...] = jnp.dot(h, w_ref[...],
                         preferred_element_type=jnp.float32) + b_ref[...]


def _tc_out(x_ref, plo_ref, phi_ref, w_ref, b_ref, wo_ref, bo_ref, o_ref):
    agg = jnp.concatenate([plo_ref[0], phi_ref[0]], axis=-1)
    h = x_ref[...] + agg
    x2 = jnp.dot(h, w_ref[...],
                 preferred_element_type=jnp.float32) + b_ref[...]
    o_ref[...] = jnp.dot(x2, wo_ref[...],
                         preferred_element_type=jnp.float32) + bo_ref[...]


def kernel(feature, edge_index, edge_type, W_in, b_in, W1, b1, W2, b2, Wo, bo):
    del edge_type
    N, D_in = feature.shape
    D = W_in.shape[1]
    Dh = D // NC
    E = edge_index.shape[1]
    BM = 1000
    grid = (N // BM,)

    nb = -(-E // (NS * BATCH))
    nb = -(-nb // NBUF) * NBUF
    e_pad = NS * BATCH * nb - E
    acc_rows = -(-(N + 1) // (NS * 8)) * (NS * 8)
    pad_dst = N
    src = edge_index[0].astype(jnp.int32)
    dst = edge_index[1].astype(jnp.int32)
    src_p = jnp.concatenate([src, jnp.zeros((e_pad,), jnp.int32)])
    src3d = jnp.stack([2 * src_p, 2 * src_p + 1]).reshape(NC, NS, nb, BATCH)
    dst3d = jnp.concatenate(
        [dst, jnp.full((e_pad,), pad_dst, jnp.int32)]).reshape(NS, nb, BATCH)

    b_in2 = b_in.reshape(1, D)
    b12 = b1.reshape(1, D)
    b22 = b2.reshape(1, D)
    Wo_p = jnp.pad(Wo, ((0, 0), (0, D - Wo.shape[1])))
    bo_p = jnp.pad(bo, (0, D - bo.shape[0])).reshape(1, D)

    row_spec = pl.BlockSpec((BM, D), lambda i: (i, 0))
    plo_spec = pl.BlockSpec((1, BM, Dh), lambda i: (0, i, 0))
    phi_spec = pl.BlockSpec((1, BM, Dh), lambda i: (1, i, 0))
    w_spec = pl.BlockSpec((D, D), lambda i: (0, 0))
    b_spec = pl.BlockSpec((1, D), lambda i: (0, 0))

    x0 = pl.pallas_call(
        _tc_in,
        grid=grid,
        in_specs=[pl.BlockSpec((BM, D_in), lambda i: (i, 0)), w_spec, b_spec],
        out_specs=row_spec,
        out_shape=jax.ShapeDtypeStruct((N, D), jnp.float32),
    )(feature, W_in, b_in2)

    p = _segment_sum_sc(x0.reshape(NC * N, Dh), src3d, dst3d, acc_rows, nb)

    x1 = pl.pallas_call(
        _tc_mid,
        grid=grid,
        in_specs=[row_spec, plo_spec, phi_spec, w_spec, b_spec],
        out_specs=row_spec,
        out_shape=jax.ShapeDtypeStruct((N, D), jnp.float32),
    )(x0, p, p, W1, b12)

    q = _segment_sum_sc(x1.reshape(NC * N, Dh), src3d, dst3d, acc_rows, nb)

    out = pl.pallas_call(
        _tc_out,
        grid=grid,
        in_specs=[row_spec, plo_spec, phi_spec, w_spec, b_spec,
                  w_spec, b_spec],
        out_specs=row_spec,
        out_shape=jax.ShapeDtypeStruct((N, D), jnp.float32),
    )(x1, q, q, W2, b22, Wo_p, bo_p)

    return out[:, :Wo.shape[1]]


# speedup vs baseline: 5.7153x; 1.0574x over previous
import functools

import jax
import jax.numpy as jnp
from jax import lax
from jax.experimental import pallas as pl
from jax.experimental.pallas import tpu as pltpu
from jax.experimental.pallas import tpu_sc as plsc

NC = 2
NS = 16
NW = NC * NS
BATCH = 56
NBUF = 2


def _segment_sum_sc(x, src4d, dst4d, acc_rows, nb):
    D = x.shape[1]
    stripe = acc_rows // NS
    mesh = plsc.VectorSubcoreMesh(core_axis_name="c", subcore_axis_name="s")

    @functools.partial(
        pl.kernel,
        out_type=jax.ShapeDtypeStruct((NC, acc_rows, D), jnp.float32),
        mesh=mesh,
        compiler_params=pltpu.CompilerParams(use_tc_tiling_on_sc=False),
        scratch_types=[
            pltpu.VMEM((nb, BATCH), jnp.int32),
            pltpu.VMEM((nb, BATCH), jnp.int32),
            pltpu.VMEM((NBUF, BATCH, D), jnp.float32),
            pltpu.VMEM_SHARED((acc_rows, D), jnp.float32),
            [pltpu.SemaphoreType.DMA] * NBUF,
        ],
    )
    def seg_kernel(x_hbm, src_hbm, dst_hbm, zero_hbm, out_hbm,
                   src_v, dst_v, rows_v, acc, sem_g):
        cid = lax.axis_index("c")
        sid = lax.axis_index("s")

        pltpu.sync_copy(src_hbm.at[cid, sid], src_v)
        pltpu.sync_copy(dst_hbm.at[cid, sid], dst_v)

        pltpu.sync_copy(zero_hbm, acc.at[pl.ds(sid * stripe, stripe)])
        plsc.subcore_barrier()

        def start_g(j, b):
            pltpu.async_copy(x_hbm.at[src_v.at[j]], rows_v.at[b], sem_g[b])

        def wait_g(j, b):
            pltpu.make_async_copy(
                x_hbm.at[src_v.at[j]], rows_v.at[b], sem_g[b]).wait()

        for b in range(NBUF):
            start_g(b, b)

        def step(i, _):
            for b in range(NBUF):
                j = i * NBUF + b
                wait_g(j, b)
                pltpu.sync_copy(rows_v.at[b], acc.at[dst_v.at[j]], add=True)

                @pl.when(j + NBUF < nb)
                def _():
                    start_g(j + NBUF, b)
            return 0

        lax.fori_loop(0, nb // NBUF, step, 0)
        plsc.subcore_barrier()

        pltpu.sync_copy(acc.at[pl.ds(sid * stripe, stripe)],
                        out_hbm.at[cid, pl.ds(sid * stripe, stripe)])

    zero = jnp.zeros((stripe, D), jnp.float32)
    return seg_kernel(x, src4d, dst4d, zero)


def _tc_in(feat_ref, w_ref, b_ref, o_ref):
    y = jnp.dot(feat_ref[...], w_ref[...],
                preferred_element_type=jnp.float32) + b_ref[...]
    o_ref[...] = jnp.where(y >= 0, y, 0.01 * y)


def _tc_mid(x_ref, p_ref, w_ref, b_ref, o_ref):
    h = x_ref[...] + p_ref[0] + p_ref[1]
    o_ref[...] = jnp.dot(h, w_ref[...],
                         preferred_element_type=jnp.float32) + b_ref[...]


def _tc_out(x_ref, p_ref, w_ref, b_ref, wo_ref, bo_ref, o_ref):
    h = x_ref[...] + p_ref[0] + p_ref[1]
    x2 = jnp.dot(h, w_ref[...],
                 preferred_element_type=jnp.float32) + b_ref[...]
    o_ref[...] = jnp.dot(x2, wo_ref[...],
                         preferred_element_type=jnp.float32) + bo_ref[...]


def kernel(feature, edge_index, edge_type, W_in, b_in, W1, b1, W2, b2, Wo, bo):
    del edge_type
    N, D_in = feature.shape
    D = W_in.shape[1]
    E = edge_index.shape[1]
    BM = 1000
    grid = (N // BM,)

    nb = -(-E // (NC * NS * BATCH))
    nb = -(-nb // NBUF) * NBUF
    e_pad = NC * NS * BATCH * nb - E
    acc_rows = -(-(N + 1) // (NS * 8)) * (NS * 8)
    pad_dst = N
    src = edge_index[0].astype(jnp.int32)
    dst = edge_index[1].astype(jnp.int32)
    src4d = jnp.concatenate(
        [src, jnp.zeros((e_pad,), jnp.int32)]).reshape(NC, NS, nb, BATCH)
    dst4d = jnp.concatenate(
        [dst, jnp.full((e_pad,), pad_dst, jnp.int32)]).reshape(NC, NS, nb, BATCH)

    b_in2 = b_in.reshape(1, D)
    b12 = b1.reshape(1, D)
    b22 = b2.reshape(1, D)
    Wo_p = jnp.pad(Wo, ((0, 0), (0, D - Wo.shape[1])))
    bo_p = jnp.pad(bo, (0, D - bo.shape[0])).reshape(1, D)

    row_spec = pl.BlockSpec((BM, D), lambda i: (i, 0))
    par_spec = pl.BlockSpec((NC, BM, D), lambda i: (0, i, 0))
    w_spec = pl.BlockSpec((D, D), lambda i: (0, 0))
    b_spec = pl.BlockSpec((1, D), lambda i: (0, 0))

    x0 = pl.pallas_call(
        _tc_in,
        grid=grid,
        in_specs=[pl.BlockSpec((BM, D_in), lambda i: (i, 0)), w_spec, b_spec],
        out_specs=row_spec,
        out_shape=jax.ShapeDtypeStruct((N, D), jnp.float32),
    )(feature, W_in, b_in2)

    p = _segment_sum_sc(x0, src4d, dst4d, acc_rows, nb)

    x1 = pl.pallas_call(
        _tc_mid,
        grid=grid,
        in_specs=[row_spec, par_spec, w_spec, b_spec],
        out_specs=row_spec,
        out_shape=jax.ShapeDtypeStruct((N, D), jnp.float32),
    )(x0, p, W1, b12)

    q = _segment_sum_sc(x1, src4d, dst4d, acc_rows, nb)

    out = pl.pallas_call(
        _tc_out,
        grid=grid,
        in_specs=[row_spec, par_spec, w_spec, b_spec, w_spec, b_spec],
        out_specs=row_spec,
        out_shape=jax.ShapeDtypeStruct((N, D), jnp.float32),
    )(x1, q, W2, b22, Wo_p, bo_p)

    return out[:, :Wo.shape[1]]


# speedup vs baseline: 6.0667x; 1.0615x over previous
import functools

import jax
import jax.numpy as jnp
from jax import lax
from jax.experimental import pallas as pl
from jax.experimental.pallas import tpu as pltpu
from jax.experimental.pallas import tpu_sc as plsc

NC = 2
NS = 16
NW = NC * NS
BATCH = 112
NBUF = 2


def _segment_sum_sc(x, src4d, dst4d, acc_rows, nb):
    D = x.shape[1]
    stripe = acc_rows // NS
    mesh = plsc.VectorSubcoreMesh(core_axis_name="c", subcore_axis_name="s")

    @functools.partial(
        pl.kernel,
        out_type=jax.ShapeDtypeStruct((NC, acc_rows, D), jnp.float32),
        mesh=mesh,
        compiler_params=pltpu.CompilerParams(use_tc_tiling_on_sc=False),
        scratch_types=[
            pltpu.VMEM((nb, BATCH), jnp.int32),
            pltpu.VMEM((nb, BATCH), jnp.int32),
            pltpu.VMEM((NBUF, BATCH, D), jnp.float32),
            pltpu.VMEM_SHARED((acc_rows, D), jnp.float32),
            [pltpu.SemaphoreType.DMA] * NBUF,
        ],
    )
    def seg_kernel(x_hbm, src_hbm, dst_hbm, zero_hbm, out_hbm,
                   src_v, dst_v, rows_v, acc, sem_g):
        cid = lax.axis_index("c")
        sid = lax.axis_index("s")

        pltpu.sync_copy(src_hbm.at[cid, sid], src_v)
        pltpu.sync_copy(dst_hbm.at[cid, sid], dst_v)

        pltpu.sync_copy(zero_hbm, acc.at[pl.ds(sid * stripe, stripe)])
        plsc.subcore_barrier()

        def start_g(j, b):
            pltpu.async_copy(x_hbm.at[src_v.at[j]], rows_v.at[b], sem_g[b])

        def wait_g(j, b):
            pltpu.make_async_copy(
                x_hbm.at[src_v.at[j]], rows_v.at[b], sem_g[b]).wait()

        for b in range(NBUF):
            start_g(b, b)

        def step(i, _):
            for b in range(NBUF):
                j = i * NBUF + b
                wait_g(j, b)
                pltpu.sync_copy(rows_v.at[b], acc.at[dst_v.at[j]], add=True)

                @pl.when(j + NBUF < nb)
                def _():
                    start_g(j + NBUF, b)
            return 0

        lax.fori_loop(0, nb // NBUF, step, 0)
        plsc.subcore_barrier()

        pltpu.sync_copy(acc.at[pl.ds(sid * stripe, stripe)],
                        out_hbm.at[cid, pl.ds(sid * stripe, stripe)])

    zero = jnp.zeros((stripe, D), jnp.float32)
    return seg_kernel(x, src4d, dst4d, zero)


def _tc_in(feat_ref, w_ref, b_ref, o_ref):
    y = jnp.dot(feat_ref[...], w_ref[...],
                preferred_element_type=jnp.float32) + b_ref[...]
    o_ref[...] = jnp.where(y >= 0, y, 0.01 * y)


def _tc_mid(x_ref, p_ref, w_ref, b_ref, o_ref):
    h = x_ref[...] + p_ref[0] + p_ref[1]
    o_ref[...] = jnp.dot(h, w_ref[...],
                         preferred_element_type=jnp.float32) + b_ref[...]


def _tc_out(x_ref, p_ref, w_ref, b_ref, wo_ref, bo_ref, o_ref):
    h = x_ref[...] + p_ref[0] + p_ref[1]
    x2 = jnp.dot(h, w_ref[...],
                 preferred_element_type=jnp.float32) + b_ref[...]
    o_ref[...] = jnp.dot(x2, wo_ref[...],
                         preferred_element_type=jnp.float32) + bo_ref[...]


def kernel(feature, edge_index, edge_type, W_in, b_in, W1, b1, W2, b2, Wo, bo):
    del edge_type
    N, D_in = feature.shape
    D = W_in.shape[1]
    E = edge_index.shape[1]
    BM = 1000
    grid = (N // BM,)

    nb = -(-E // (NC * NS * BATCH))
    nb = -(-nb // NBUF) * NBUF
    e_pad = NC * NS * BATCH * nb - E
    acc_rows = -(-(N + 1) // (NS * 8)) * (NS * 8)
    pad_dst = N
    src = edge_index[0].astype(jnp.int32)
    dst = edge_index[1].astype(jnp.int32)
    src4d = jnp.concatenate(
        [src, jnp.zeros((e_pad,), jnp.int32)]).reshape(NC, NS, nb, BATCH)
    dst4d = jnp.concatenate(
        [dst, jnp.full((e_pad,), pad_dst, jnp.int32)]).reshape(NC, NS, nb, BATCH)

    b_in2 = b_in.reshape(1, D)
    b12 = b1.reshape(1, D)
    b22 = b2.reshape(1, D)
    Wo_p = jnp.pad(Wo, ((0, 0), (0, D - Wo.shape[1])))
    bo_p = jnp.pad(bo, (0, D - bo.shape[0])).reshape(1, D)

    row_spec = pl.BlockSpec((BM, D), lambda i: (i, 0))
    par_spec = pl.BlockSpec((NC, BM, D), lambda i: (0, i, 0))
    w_spec = pl.BlockSpec((D, D), lambda i: (0, 0))
    b_spec = pl.BlockSpec((1, D), lambda i: (0, 0))

    x0 = pl.pallas_call(
        _tc_in,
        grid=grid,
        in_specs=[pl.BlockSpec((BM, D_in), lambda i: (i, 0)), w_spec, b_spec],
        out_specs=row_spec,
        out_shape=jax.ShapeDtypeStruct((N, D), jnp.float32),
    )(feature, W_in, b_in2)

    p = _segment_sum_sc(x0, src4d, dst4d, acc_rows, nb)

    x1 = pl.pallas_call(
        _tc_mid,
        grid=grid,
        in_specs=[row_spec, par_spec, w_spec, b_spec],
        out_specs=row_spec,
        out_shape=jax.ShapeDtypeStruct((N, D), jnp.float32),
    )(x0, p, W1, b12)

    q = _segment_sum_sc(x1, src4d, dst4d, acc_rows, nb)

    out = pl.pallas_call(
        _tc_out,
        grid=grid,
        in_specs=[row_spec, par_spec, w_spec, b_spec, w_spec, b_spec],
        out_specs=row_spec,
        out_shape=jax.ShapeDtypeStruct((N, D), jnp.float32),
    )(x1, q, W2, b22, Wo_p, bo_p)

    return out[:, :Wo.shape[1]]
